# Initial kernel scaffold; baseline (speedup 1.0000x reference)
import jax
import jax.numpy as jnp
from jax.experimental import pallas as pl


def kernel(input_ids, reg_table, pos_table, gamma, beta):
    raise NotImplementedError("write your pallas kernel here")


# speedup vs baseline: 1.0183x; 1.0183x over previous
import functools

import jax
import jax.numpy as jnp
from jax import lax
from jax.experimental import pallas as pl
from jax.experimental.pallas import tpu as pltpu
from jax.experimental.pallas import tpu_sc as plsc

D = 128
LANES = 16
NJ = D // LANES
CHUNK = 100
B, L = 1024, 200
NROWS = B * L // CHUNK
NC, NS = 2, 16
NW = NC * NS
NPW = NROWS // NW
PER_SEQ = L // CHUNK


def _lane_bcast_sum(x):
    idx0 = lax.iota(jnp.int32, LANES)
    dnums = lax.GatherDimensionNumbers(
        offset_dims=(), collapsed_slice_dims=(0,), start_index_map=(0,))
    for s in (8, 4, 2, 1):
        idx = lax.bitwise_and(idx0 + s, LANES - 1)
        x = x + lax.gather(x, idx[:, None], dnums, slice_sizes=(1,),
                           mode=lax.GatherScatterMode.PROMISE_IN_BOUNDS)
    return x


def _sc_body(ids_hbm, table_hbm, pos_hbm, gamma_hbm, beta_hbm, out_hbm,
             idx_v, rows_v, pos_v, g_v, b_v, sem):
    wid = lax.axis_index("c") * NS + lax.axis_index("s")

    pltpu.sync_copy(pos_hbm.at[pl.ds(0, L)], pos_v)
    pltpu.sync_copy(gamma_hbm, g_v)
    pltpu.sync_copy(beta_hbm, b_v)

    def chunk_body(g, carry):
        r = wid * NPW + g
        pltpu.sync_copy(ids_hbm.at[r], idx_v)
        pltpu.async_copy(table_hbm.at[idx_v], rows_v, sem).wait()
        pbase = lax.rem(r, PER_SEQ) * CHUNK

        def row_body(i, c2):
            x = [rows_v[i, pl.ds(LANES * j, LANES)] +
                 pos_v[pbase + i, pl.ds(LANES * j, LANES)] for j in range(NJ)]
            acc = x[0]
            for j in range(1, NJ):
                acc = acc + x[j]
            mv = _lane_bcast_sum(acc) * (1.0 / D)
            c = [xj - mv for xj in x]
            acc2 = c[0] * c[0]
            for j in range(1, NJ):
                acc2 = acc2 + c[j] * c[j]
            vv = _lane_bcast_sum(acc2) * (1.0 / D) + 1e-12
            iv = lax.bitcast_convert_type(vv, jnp.int32)
            iv = jnp.int32(0x5F3759DF) - lax.shift_right_logical(iv, 1)
            y = lax.bitcast_convert_type(iv, jnp.float32)
            for _ in range(3):
                y = y * (1.5 - 0.5 * vv * y * y)
            for j in range(NJ):
                rows_v[i, pl.ds(LANES * j, LANES)] = (
                    c[j] * y * g_v[pl.ds(LANES * j, LANES)]
                    + b_v[pl.ds(LANES * j, LANES)])
            return c2

        lax.fori_loop(0, CHUNK, row_body, 0)
        pltpu.sync_copy(rows_v, out_hbm.at[r])
        return carry

    lax.fori_loop(0, NPW, chunk_body, 0)


@jax.jit
def _run(ids2, reg_table, pos_table, gamma, beta):
    call = functools.partial(
        pl.kernel,
        mesh=plsc.VectorSubcoreMesh(core_axis_name="c", subcore_axis_name="s"),
        out_type=jax.ShapeDtypeStruct((NROWS, CHUNK, D), jnp.float32),
        scratch_types=[
            pltpu.VMEM((CHUNK,), jnp.int32),
            pltpu.VMEM((CHUNK, D), jnp.float32),
            pltpu.VMEM((L, D), jnp.float32),
            pltpu.VMEM((D,), jnp.float32),
            pltpu.VMEM((D,), jnp.float32),
            pltpu.SemaphoreType.DMA,
        ],
    )(_sc_body)
    return call(ids2, reg_table, pos_table, gamma, beta)


def kernel(input_ids, reg_table, pos_table, gamma, beta):
    ids2 = input_ids.astype(jnp.int32).reshape(NROWS, CHUNK)
    out = _run(ids2, reg_table, pos_table, gamma, beta)
    return out.reshape(B, L, D)
